# baseline (speedup 1.0000x reference)
import jax
import jax.numpy as jnp
from jax.experimental import pallas as pl
from jax.experimental.pallas import tpu as pltpu


def _identity_kernel(x_ref, o_ref):
    o_ref[...] = x_ref[...]


def kernel(x, W):
    n, d = x.shape
    dp = 32
    xp = jnp.pad(x, ((0, 0), (0, dp - d)))
    xr = jnp.reshape(xp, (n * dp // 128, 128))
    m = xr.shape[0]
    out = pl.pallas_call(
        _identity_kernel,
        grid=(2,),
        in_specs=[pl.BlockSpec((m // 2, 128), lambda i: (i, 0))],
        out_specs=pl.BlockSpec((m // 2, 128), lambda i: (i, 0)),
        out_shape=jax.ShapeDtypeStruct(xr.shape, xr.dtype),
        compiler_params=pltpu.CompilerParams(allow_input_fusion=[True]),
    )(xr)
    return jnp.reshape(out, (n, dp))[:, :d]
